# baseline (speedup 1.0000x reference)
import jax
import jax.numpy as jnp
from jax.experimental import pallas as pl
from jax.experimental.pallas import tpu as pltpu

B = 16
D = 64
HW = 1024
N = 1024
BETA = 0.25


def _vq_body(z_ref, w_ref, zq_ref, idx_ref, acc_ref, div_ref, iota_ref):
    b = pl.program_id(0)
    zc = z_ref[0]
    w = w_ref[...]
    wsq = jnp.sum(w * w, axis=1, keepdims=True)
    dots2 = jax.lax.dot_general(
        -2.0 * w, zc, (((1,), (0,)), ((), ())),
        preferred_element_type=jnp.float32)
    dist_t = wsq + dots2
    min_d = jnp.min(dist_t, axis=0, keepdims=True)
    @pl.when(b == 0)
    def _():
        iota_ref[...] = jax.lax.broadcasted_iota(
            jnp.int32, (N, HW), 0).astype(jnp.float32)

    iota_f = iota_ref[...]
    sel = jnp.where(dist_t == min_d, iota_f, jnp.float32(N))
    idxf = jnp.min(sel, axis=0)
    idx = idxf.astype(jnp.int32)
    idx_ref[0, 0] = idx
    ohf = (iota_f == idxf[None, :]).astype(jnp.float32)
    zq = jax.lax.dot_general(
        w, ohf, (((0,), (0,)), ((), ())),
        preferred_element_type=jnp.float32)
    zq_ref[0] = zq
    cnts = jax.lax.dot_general(
        ohf, jnp.ones((HW, 128), jnp.float32),
        (((1,), (0,)), ((), ())),
        preferred_element_type=jnp.float32)
    usedf = (cnts[:, 0:1] > 0.0).astype(jnp.float32)
    val = jnp.sum(min_d) + jnp.sum(zc * zc)
    dval = jnp.sum(usedf)

    @pl.when(b == 0)
    def _():
        acc_ref[0, 0] = val
        div_ref[0, 0] = dval

    @pl.when(b > 0)
    def _():
        acc_ref[0, 0] += val
        div_ref[0, 0] += dval


def kernel(z, weight):
    zr = z.reshape(B, D, HW)
    zq, idx, acc, div = pl.pallas_call(
        _vq_body,
        grid=(B,),
        in_specs=[
            pl.BlockSpec((1, D, HW), lambda b: (b, 0, 0)),
            pl.BlockSpec((N, D), lambda b: (0, 0)),
        ],
        out_specs=[
            pl.BlockSpec((1, D, HW), lambda b: (b, 0, 0)),
            pl.BlockSpec((1, 1, HW), lambda b: (b, 0, 0)),
            pl.BlockSpec(memory_space=pltpu.SMEM),
            pl.BlockSpec(memory_space=pltpu.SMEM),
        ],
        out_shape=[
            jax.ShapeDtypeStruct((B, D, HW), jnp.float32),
            jax.ShapeDtypeStruct((B, 1, HW), jnp.int32),
            jax.ShapeDtypeStruct((1, 1), jnp.float32),
            jax.ShapeDtypeStruct((1, 1), jnp.float32),
        ],
        scratch_shapes=[pltpu.VMEM((N, HW), jnp.float32)],
        compiler_params=pltpu.CompilerParams(
            dimension_semantics=("arbitrary",),
        ),
    )(zr, weight)
    z_q_out = zq.reshape(B, D, 32, 32)
    index = idx.reshape(B, 32, 32)
    loss = acc[0, 0] * ((1.0 + BETA) / (B * HW * D))
    diversity = div[0, 0] / (B * HW)
    return z_q_out, index, loss, diversity


# speedup vs baseline: 1.1300x; 1.1300x over previous
import jax
import jax.numpy as jnp
from jax.experimental import pallas as pl
from jax.experimental.pallas import tpu as pltpu

B = 16
D = 64
HW = 1024
N = 1024
BETA = 0.25


def _vq_body(z_ref, w_ref, zq_ref, idx_ref, acc_ref, div_ref):
    b = pl.program_id(0)
    zc = z_ref[0]
    w = w_ref[...]
    wsq = jnp.sum(w * w, axis=1, keepdims=True)
    dots2 = jax.lax.dot_general(
        -2.0 * w, zc, (((1,), (0,)), ((), ())),
        preferred_element_type=jnp.float32)
    dist_t = wsq + dots2
    min_d = jnp.min(dist_t, axis=0, keepdims=True)
    iota_t = jax.lax.broadcasted_iota(jnp.int32, (N, HW), 0)
    idx = jnp.min(jnp.where(dist_t == min_d, iota_t, N), axis=0)
    idx_ref[0, 0] = idx
    ohf = (iota_t == idx[None, :]).astype(jnp.float32)
    zq = jax.lax.dot_general(
        w, ohf, (((0,), (0,)), ((), ())),
        preferred_element_type=jnp.float32)
    zq_ref[0] = zq
    cnts = jax.lax.dot_general(
        ohf, jnp.ones((HW, 128), jnp.float32),
        (((1,), (0,)), ((), ())),
        preferred_element_type=jnp.float32)
    usedf = (cnts[:, 0:1] > 0.0).astype(jnp.float32)
    val = jnp.sum(min_d) + jnp.sum(zc * zc)
    dval = jnp.sum(usedf)

    @pl.when(b == 0)
    def _():
        acc_ref[0, 0] = val
        div_ref[0, 0] = dval

    @pl.when(b > 0)
    def _():
        acc_ref[0, 0] += val
        div_ref[0, 0] += dval


def kernel(z, weight):
    zr = z.reshape(B, D, HW)
    zq, idx, acc, div = pl.pallas_call(
        _vq_body,
        grid=(B,),
        in_specs=[
            pl.BlockSpec((1, D, HW), lambda b: (b, 0, 0)),
            pl.BlockSpec((N, D), lambda b: (0, 0)),
        ],
        out_specs=[
            pl.BlockSpec((1, D, HW), lambda b: (b, 0, 0)),
            pl.BlockSpec((1, 1, HW), lambda b: (b, 0, 0)),
            pl.BlockSpec(memory_space=pltpu.SMEM),
            pl.BlockSpec(memory_space=pltpu.SMEM),
        ],
        out_shape=[
            jax.ShapeDtypeStruct((B, D, HW), jnp.float32),
            jax.ShapeDtypeStruct((B, 1, HW), jnp.int32),
            jax.ShapeDtypeStruct((1, 1), jnp.float32),
            jax.ShapeDtypeStruct((1, 1), jnp.float32),
        ],
        compiler_params=pltpu.CompilerParams(
            dimension_semantics=("arbitrary",),
        ),
    )(zr, weight)
    z_q_out = zq.reshape(B, D, 32, 32)
    index = idx.reshape(B, 32, 32)
    loss = acc[0, 0] * ((1.0 + BETA) / (B * HW * D))
    diversity = div[0, 0] / (B * HW)
    return z_q_out, index, loss, diversity
